# baseline (speedup 1.0000x reference)
import functools

import jax
import jax.numpy as jnp
from jax import lax
from jax.experimental import pallas as pl
from jax.experimental.pallas import tpu as pltpu
from jax.experimental.pallas import tpu_sc as plsc

VOC_SIZE = 1000000
EMB_SIZE = 16
BATCH = 16384

_NC = 2
_NS = 16
_NW = _NC * _NS
_BPW = BATCH // _NW
_G = 4
_NSET = 4
_GW = _G * 128
_NI = _BPW // (_G * _NSET)
_WMAX = VOC_SIZE - 128


def _win_base_vec(v):
    c = lax.shift_left(lax.shift_right_logical(v, 7), 7)
    return jnp.minimum(c, _WMAX)


def _glove_body(center_hbm, outside_hbm, coocs_hbm, w_hbm,
                ceT_hbm, oeT_hbm, cb_hbm, ob_hbm, out_hbm,
                cidx_v, oidx_v, cooc_v, wv_v,
                cew0, cew1, cew2, cew3, oew0, oew1, oew2, oew3,
                cbw0, cbw1, cbw2, cbw3, obw0, obw1, obw2, obw3,
                out_v, sem0, sem1, sem2, sem3):
    wid = lax.axis_index("s") * _NC + lax.axis_index("c")
    base = wid * _BPW

    pltpu.sync_copy(center_hbm.at[pl.ds(base, _BPW)], cidx_v)
    pltpu.sync_copy(outside_hbm.at[pl.ds(base, _BPW)], oidx_v)
    pltpu.sync_copy(coocs_hbm.at[pl.ds(base, _BPW)], cooc_v)
    pltpu.sync_copy(w_hbm.at[pl.ds(base, _BPW)], wv_v)

    cews = (cew0, cew1, cew2, cew3)
    oews = (oew0, oew1, oew2, oew3)
    cbws = (cbw0, cbw1, cbw2, cbw3)
    obws = (obw0, obw1, obw2, obw3)
    sems = (sem0, sem1, sem2, sem3)

    lane = lax.broadcasted_iota(jnp.int32, (16,), 0)

    def fire(cvv, cuv, l0, s):
        for j in range(_G):
            cv = pl.multiple_of(cvv[l0 + j], 128)
            cu = pl.multiple_of(cuv[l0 + j], 128)
            pltpu.make_async_copy(
                ceT_hbm.at[:, pl.ds(cv, 128)],
                cews[s].at[:, pl.ds(j * 128, 128)], sems[s]).start()
            pltpu.make_async_copy(
                oeT_hbm.at[:, pl.ds(cu, 128)],
                oews[s].at[:, pl.ds(j * 128, 128)], sems[s]).start()
            pltpu.make_async_copy(cb_hbm.at[:, pl.ds(cv, 128)],
                                  cbws[s].at[pl.ds(j, 1), :], sems[s]).start()
            pltpu.make_async_copy(ob_hbm.at[:, pl.ds(cu, 128)],
                                  obws[s].at[pl.ds(j, 1), :], sems[s]).start()

    def drain(s):
        pltpu.make_async_copy(ceT_hbm.at[:, pl.ds(0, _GW)],
                              cews[s], sems[s]).wait()
        pltpu.make_async_copy(oeT_hbm.at[:, pl.ds(0, _GW)],
                              oews[s], sems[s]).wait()
        pltpu.make_async_copy(ceT_hbm.at[pl.ds(0, _G), pl.ds(0, 128)],
                              cbws[s], sems[s]).wait()
        pltpu.make_async_copy(ceT_hbm.at[pl.ds(0, _G), pl.ds(0, 128)],
                              obws[s], sems[s]).wait()

    def compute(lvv, luv, cvec, wvec, l0, s, acc):
        for j in range(_G):
            lv16 = jnp.broadcast_to(lvv[l0 + j], (16,))
            lu16 = jnp.broadcast_to(luv[l0 + j], (16,))
            j16 = jnp.full((16,), j, jnp.int32)
            ce = plsc.load_gather(cews[s], [lane, j * 128 + lv16])
            oe = plsc.load_gather(oews[s], [lane, j * 128 + lu16])
            ip = jnp.sum(ce * oe)
            cb = plsc.load_gather(cbws[s], [j16, lv16])[0]
            tb = plsc.load_gather(obws[s], [j16, lu16])[0]
            r = ip + cb + tb - cvec[l0 + j]
            acc = acc + wvec[l0 + j] * r * r
        return acc

    vc0 = cidx_v[pl.ds(0, 16)]
    vo0 = oidx_v[pl.ds(0, 16)]
    cv0 = _win_base_vec(vc0)
    cu0 = _win_base_vec(vo0)
    fire(cv0, cu0, 0, 0)
    fire(cv0, cu0, 4, 1)
    fire(cv0, cu0, 8, 2)

    def iter_body(i, carry):
        vc, vo, cv, cu, acc = carry
        inext = jnp.minimum(i + 1, _NI - 1) * 16
        vcn = cidx_v[pl.ds(inext, 16)]
        von = oidx_v[pl.ds(inext, 16)]
        cvn = _win_base_vec(vcn)
        cun = _win_base_vec(von)
        lvv = vc - cv
        luv = vo - cu
        cvec = cooc_v[pl.ds(i * 16, 16)]
        wvec = wv_v[pl.ds(i * 16, 16)]
        last = i >= _NI - 1

        drain(0)
        acc = compute(lvv, luv, cvec, wvec, 0, 0, acc)
        fire(cv, cu, 12, 3)

        drain(1)
        acc = compute(lvv, luv, cvec, wvec, 4, 1, acc)

        @pl.when(jnp.logical_not(last))
        def _():
            fire(cvn, cun, 0, 0)

        drain(2)
        acc = compute(lvv, luv, cvec, wvec, 8, 2, acc)

        @pl.when(jnp.logical_not(last))
        def _():
            fire(cvn, cun, 4, 1)

        drain(3)
        acc = compute(lvv, luv, cvec, wvec, 12, 3, acc)

        @pl.when(jnp.logical_not(last))
        def _():
            fire(cvn, cun, 8, 2)

        return (vcn, von, cvn, cun, acc)

    _, _, _, _, acc = lax.fori_loop(
        0, _NI, iter_body, (vc0, vo0, cv0, cu0, jnp.float32(0.0)))

    out_v[...] = jnp.where(lane == 0, acc, 0.0)
    pltpu.sync_copy(out_v, out_hbm.at[wid])


def _glove_partials(center, outside, coocs, weighting, ceT, oeT, cb, ob):
    mesh = plsc.VectorSubcoreMesh(core_axis_name="c", subcore_axis_name="s")
    emb_win = pltpu.VMEM((EMB_SIZE, _GW), jnp.float32)
    bias_win = pltpu.VMEM((_G, 128), jnp.float32)
    k = functools.partial(
        pl.kernel,
        mesh=mesh,
        out_type=jax.ShapeDtypeStruct((_NW, 16), jnp.float32),
        scratch_types=[
            pltpu.VMEM((_BPW,), jnp.int32),
            pltpu.VMEM((_BPW,), jnp.int32),
            pltpu.VMEM((_BPW,), jnp.float32),
            pltpu.VMEM((_BPW,), jnp.float32),
            emb_win, emb_win, emb_win, emb_win,
            emb_win, emb_win, emb_win, emb_win,
            bias_win, bias_win, bias_win, bias_win,
            bias_win, bias_win, bias_win, bias_win,
            pltpu.VMEM((16,), jnp.float32),
            pltpu.SemaphoreType.DMA,
            pltpu.SemaphoreType.DMA,
            pltpu.SemaphoreType.DMA,
            pltpu.SemaphoreType.DMA,
        ],
        compiler_params=pltpu.CompilerParams(
            needs_layout_passes=False,
            use_tc_tiling_on_sc=True,
        ),
    )(_glove_body)
    return k(center, outside, coocs, weighting, ceT, oeT, cb, ob)


def kernel(center, outside, coocs, weighting, center_embedding,
           outside_embedding, center_bias, outside_bias):
    parts = _glove_partials(
        center.reshape(-1), outside.reshape(-1),
        coocs.reshape(-1), weighting.reshape(-1),
        center_embedding.T, outside_embedding.T,
        center_bias.T, outside_bias.T,
    )
    return jnp.sum(parts)


# speedup vs baseline: 1.0167x; 1.0167x over previous
import functools

import jax
import jax.numpy as jnp
from jax import lax
from jax.experimental import pallas as pl
from jax.experimental.pallas import tpu as pltpu
from jax.experimental.pallas import tpu_sc as plsc

VOC_SIZE = 1000000
EMB_SIZE = 16
BATCH = 16384

_NC = 2
_NS = 16
_NW = _NC * _NS
_BPW = BATCH // _NW
_G = 8
_NG = _BPW // _G
_GW = _G * 128
_WMAX = VOC_SIZE - 128


def _win_base_vec(v):
    c = lax.shift_left(lax.shift_right_logical(v, 7), 7)
    return jnp.minimum(c, _WMAX)


def _glove_body(center_hbm, outside_hbm, coocs_hbm, w_hbm,
                ceT_hbm, oeT_hbm, cb_hbm, ob_hbm, out_hbm,
                cidx_v, oidx_v, cooc_v, wv_v,
                cewA, cewB, oewA, oewB, cbwA, cbwB, obwA, obwB,
                out_v, semA, semB):
    wid = lax.axis_index("s") * _NC + lax.axis_index("c")
    base = wid * _BPW

    pltpu.sync_copy(center_hbm.at[pl.ds(base, _BPW)], cidx_v)
    pltpu.sync_copy(outside_hbm.at[pl.ds(base, _BPW)], oidx_v)
    pltpu.sync_copy(coocs_hbm.at[pl.ds(base, _BPW)], cooc_v)
    pltpu.sync_copy(w_hbm.at[pl.ds(base, _BPW)], wv_v)

    lane = lax.broadcasted_iota(jnp.int32, (16,), 0)
    zeros16 = jnp.zeros((16,), jnp.int32)

    def fire(cvv, cuv, l0, cew, oew, cbw, obw, sem):
        for j in range(_G):
            cv = pl.multiple_of(cvv[l0 + j], 128)
            cu = pl.multiple_of(cuv[l0 + j], 128)
            pltpu.make_async_copy(
                ceT_hbm.at[:, pl.ds(cv, 128)],
                cew.at[:, pl.ds(j * 128, 128)], sem).start()
            pltpu.make_async_copy(
                oeT_hbm.at[:, pl.ds(cu, 128)],
                oew.at[:, pl.ds(j * 128, 128)], sem).start()
            pltpu.make_async_copy(cb_hbm.at[:, pl.ds(cv, 128)],
                                  cbw.at[pl.ds(j, 1), :], sem).start()
            pltpu.make_async_copy(ob_hbm.at[:, pl.ds(cu, 128)],
                                  obw.at[pl.ds(j, 1), :], sem).start()

    def drain(cew, oew, cbw, obw, sem):
        pltpu.make_async_copy(ceT_hbm.at[:, pl.ds(0, _GW)], cew, sem).wait()
        pltpu.make_async_copy(oeT_hbm.at[:, pl.ds(0, _GW)], oew, sem).wait()
        pltpu.make_async_copy(ceT_hbm.at[pl.ds(0, _G), pl.ds(0, 128)],
                              cbw, sem).wait()
        pltpu.make_async_copy(ceT_hbm.at[pl.ds(0, _G), pl.ds(0, 128)],
                              obw, sem).wait()

    halfj = lane & 7

    def compute(i, l0, cew, oew, cbw, obw, acc):
        off16 = i * 16 + l0 + halfj
        vh = plsc.load_gather(cidx_v, [off16])
        uh = plsc.load_gather(oidx_v, [off16])
        lvh = vh - _win_base_vec(vh)
        luh = uh - _win_base_vec(uh)
        colv = halfj * 128 + lvh
        colu = halfj * 128 + luh
        ip = jnp.zeros((16,), jnp.float32)
        for e in range(EMB_SIZE):
            e16 = jnp.full((16,), e, jnp.int32)
            ce = plsc.load_gather(cew, [e16, colv])
            oe = plsc.load_gather(oew, [e16, colu])
            ip = ip + ce * oe
        cb = plsc.load_gather(cbw, [halfj, lvh])
        tb = plsc.load_gather(obw, [halfj, luh])
        cooc = plsc.load_gather(cooc_v, [off16])
        w = plsc.load_gather(wv_v, [off16])
        r = ip + cb + tb - cooc
        return acc + jnp.where(lane < 8, w * r * r, 0.0)

    vc0 = cidx_v[pl.ds(0, 16)]
    vo0 = oidx_v[pl.ds(0, 16)]
    cv0 = _win_base_vec(vc0)
    cu0 = _win_base_vec(vo0)
    fire(cv0, cu0, 0, cewA, oewA, cbwA, obwA, semA)
    fire(cv0, cu0, 8, cewB, oewB, cbwB, obwB, semB)

    def pair_body(p, carry):
        vc, vo, cv, cu, acc = carry
        pnext = jnp.minimum(p + 1, _NG // 2 - 1) * 16
        vcn = cidx_v[pl.ds(pnext, 16)]
        von = oidx_v[pl.ds(pnext, 16)]
        cvn = _win_base_vec(vcn)
        cun = _win_base_vec(von)
        drain(cewA, oewA, cbwA, obwA, semA)
        acc = compute(p, 0, cewA, oewA, cbwA, obwA, acc)

        @pl.when(p < _NG // 2 - 1)
        def _():
            fire(cvn, cun, 0, cewA, oewA, cbwA, obwA, semA)

        drain(cewB, oewB, cbwB, obwB, semB)
        acc = compute(p, 8, cewB, oewB, cbwB, obwB, acc)

        @pl.when(p < _NG // 2 - 1)
        def _():
            fire(cvn, cun, 8, cewB, oewB, cbwB, obwB, semB)

        return (vcn, von, cvn, cun, acc)

    _, _, _, _, acc = lax.fori_loop(
        0, _NG // 2, pair_body,
        (vc0, vo0, cv0, cu0, jnp.zeros((16,), jnp.float32)))

    out_v[...] = acc
    pltpu.sync_copy(out_v, out_hbm.at[wid])


def _glove_partials(center, outside, coocs, weighting, ceT, oeT, cb, ob):
    mesh = plsc.VectorSubcoreMesh(core_axis_name="c", subcore_axis_name="s")
    k = functools.partial(
        pl.kernel,
        mesh=mesh,
        out_type=jax.ShapeDtypeStruct((_NW, 16), jnp.float32),
        scratch_types=[
            pltpu.VMEM((_BPW,), jnp.int32),
            pltpu.VMEM((_BPW,), jnp.int32),
            pltpu.VMEM((_BPW,), jnp.float32),
            pltpu.VMEM((_BPW,), jnp.float32),
            pltpu.VMEM((EMB_SIZE, _GW), jnp.float32),
            pltpu.VMEM((EMB_SIZE, _GW), jnp.float32),
            pltpu.VMEM((EMB_SIZE, _GW), jnp.float32),
            pltpu.VMEM((EMB_SIZE, _GW), jnp.float32),
            pltpu.VMEM((_G, 128), jnp.float32),
            pltpu.VMEM((_G, 128), jnp.float32),
            pltpu.VMEM((_G, 128), jnp.float32),
            pltpu.VMEM((_G, 128), jnp.float32),
            pltpu.VMEM((16,), jnp.float32),
            pltpu.SemaphoreType.DMA,
            pltpu.SemaphoreType.DMA,
        ],
        compiler_params=pltpu.CompilerParams(
            needs_layout_passes=False,
            use_tc_tiling_on_sc=True,
        ),
    )(_glove_body)
    return k(center, outside, coocs, weighting, ceT, oeT, cb, ob)


def kernel(center, outside, coocs, weighting, center_embedding,
           outside_embedding, center_bias, outside_bias):
    parts = _glove_partials(
        center.reshape(-1), outside.reshape(-1),
        coocs.reshape(-1), weighting.reshape(-1),
        center_embedding.T, outside_embedding.T,
        center_bias.T, outside_bias.T,
    )
    return jnp.sum(parts)


# speedup vs baseline: 1.0343x; 1.0173x over previous
import functools

import jax
import jax.numpy as jnp
from jax import lax
from jax.experimental import pallas as pl
from jax.experimental.pallas import tpu as pltpu
from jax.experimental.pallas import tpu_sc as plsc

VOC_SIZE = 1000000
EMB_SIZE = 16
BATCH = 16384

_NC = 2
_NS = 16
_NW = _NC * _NS
_BPW = BATCH // _NW
_G = 8
_NG = _BPW // _G
_GW = _G * 128
_WMAX = VOC_SIZE - 128


def _win_base_vec(v):
    c = lax.shift_left(lax.shift_right_logical(v, 7), 7)
    return jnp.minimum(c, _WMAX)


def _glove_body(center_hbm, outside_hbm, coocs_hbm, w_hbm,
                ceT_hbm, oeT_hbm, cb_hbm, ob_hbm, out_hbm,
                cidx_v, oidx_v, cooc_v, wv_v,
                cewA, cewB, oewA, oewB, cbwA, cbwB, obwA, obwB,
                out_v, semA, semB):
    wid = lax.axis_index("s") * _NC + lax.axis_index("c")
    base = wid * _BPW

    pltpu.sync_copy(center_hbm.at[pl.ds(base, _BPW)], cidx_v)
    pltpu.sync_copy(outside_hbm.at[pl.ds(base, _BPW)], oidx_v)
    pltpu.sync_copy(coocs_hbm.at[pl.ds(base, _BPW)], cooc_v)
    pltpu.sync_copy(w_hbm.at[pl.ds(base, _BPW)], wv_v)

    lane = lax.broadcasted_iota(jnp.int32, (16,), 0)
    zeros16 = jnp.zeros((16,), jnp.int32)

    def fire(cvv, cuv, l0, cew, oew, cbw, obw, sem):
        for j in range(_G):
            cv = pl.multiple_of(cvv[l0 + j], 128)
            cu = pl.multiple_of(cuv[l0 + j], 128)
            pltpu.make_async_copy(
                ceT_hbm.at[pl.ds(0, 8), pl.ds(cv, 128)],
                cew.at[pl.ds(0, 8), pl.ds(j * 128, 128)], sem).start()
            pltpu.make_async_copy(
                ceT_hbm.at[pl.ds(8, 8), pl.ds(cv, 128)],
                cew.at[pl.ds(8, 8), pl.ds(j * 128, 128)], sem).start()
            pltpu.make_async_copy(
                oeT_hbm.at[pl.ds(0, 8), pl.ds(cu, 128)],
                oew.at[pl.ds(0, 8), pl.ds(j * 128, 128)], sem).start()
            pltpu.make_async_copy(
                oeT_hbm.at[pl.ds(8, 8), pl.ds(cu, 128)],
                oew.at[pl.ds(8, 8), pl.ds(j * 128, 128)], sem).start()
            pltpu.make_async_copy(cb_hbm.at[:, pl.ds(cv, 128)],
                                  cbw.at[pl.ds(j, 1), :], sem).start()
            pltpu.make_async_copy(ob_hbm.at[:, pl.ds(cu, 128)],
                                  obw.at[pl.ds(j, 1), :], sem).start()

    def drain(cew, oew, cbw, obw, sem):
        pltpu.make_async_copy(ceT_hbm.at[:, pl.ds(0, _GW)], cew, sem).wait()
        pltpu.make_async_copy(oeT_hbm.at[:, pl.ds(0, _GW)], oew, sem).wait()
        pltpu.make_async_copy(ceT_hbm.at[pl.ds(0, _G), pl.ds(0, 128)],
                              cbw, sem).wait()
        pltpu.make_async_copy(ceT_hbm.at[pl.ds(0, _G), pl.ds(0, 128)],
                              obw, sem).wait()

    halfj = lane & 7

    def compute(i, l0, cew, oew, cbw, obw, acc):
        off16 = i * 16 + l0 + halfj
        vh = plsc.load_gather(cidx_v, [off16])
        uh = plsc.load_gather(oidx_v, [off16])
        lvh = vh - _win_base_vec(vh)
        luh = uh - _win_base_vec(uh)
        colv = halfj * 128 + lvh
        colu = halfj * 128 + luh
        ip = jnp.zeros((16,), jnp.float32)
        for e in range(EMB_SIZE):
            e16 = jnp.full((16,), e, jnp.int32)
            ce = plsc.load_gather(cew, [e16, colv])
            oe = plsc.load_gather(oew, [e16, colu])
            ip = ip + ce * oe
        cb = plsc.load_gather(cbw, [halfj, lvh])
        tb = plsc.load_gather(obw, [halfj, luh])
        cooc = plsc.load_gather(cooc_v, [off16])
        w = plsc.load_gather(wv_v, [off16])
        r = ip + cb + tb - cooc
        return acc + jnp.where(lane < 8, w * r * r, 0.0)

    vc0 = cidx_v[pl.ds(0, 16)]
    vo0 = oidx_v[pl.ds(0, 16)]
    cv0 = _win_base_vec(vc0)
    cu0 = _win_base_vec(vo0)
    fire(cv0, cu0, 0, cewA, oewA, cbwA, obwA, semA)
    fire(cv0, cu0, 8, cewB, oewB, cbwB, obwB, semB)

    def pair_body(p, carry):
        vc, vo, cv, cu, acc = carry
        pnext = jnp.minimum(p + 1, _NG // 2 - 1) * 16
        vcn = cidx_v[pl.ds(pnext, 16)]
        von = oidx_v[pl.ds(pnext, 16)]
        cvn = _win_base_vec(vcn)
        cun = _win_base_vec(von)
        drain(cewA, oewA, cbwA, obwA, semA)
        acc = compute(p, 0, cewA, oewA, cbwA, obwA, acc)

        @pl.when(p < _NG // 2 - 1)
        def _():
            fire(cvn, cun, 0, cewA, oewA, cbwA, obwA, semA)

        drain(cewB, oewB, cbwB, obwB, semB)
        acc = compute(p, 8, cewB, oewB, cbwB, obwB, acc)

        @pl.when(p < _NG // 2 - 1)
        def _():
            fire(cvn, cun, 8, cewB, oewB, cbwB, obwB, semB)

        return (vcn, von, cvn, cun, acc)

    _, _, _, _, acc = lax.fori_loop(
        0, _NG // 2, pair_body,
        (vc0, vo0, cv0, cu0, jnp.zeros((16,), jnp.float32)))

    out_v[...] = acc
    pltpu.sync_copy(out_v, out_hbm.at[wid])


def _glove_partials(center, outside, coocs, weighting, ceT, oeT, cb, ob):
    mesh = plsc.VectorSubcoreMesh(core_axis_name="c", subcore_axis_name="s")
    k = functools.partial(
        pl.kernel,
        mesh=mesh,
        out_type=jax.ShapeDtypeStruct((_NW, 16), jnp.float32),
        scratch_types=[
            pltpu.VMEM((_BPW,), jnp.int32),
            pltpu.VMEM((_BPW,), jnp.int32),
            pltpu.VMEM((_BPW,), jnp.float32),
            pltpu.VMEM((_BPW,), jnp.float32),
            pltpu.VMEM((EMB_SIZE, _GW), jnp.float32),
            pltpu.VMEM((EMB_SIZE, _GW), jnp.float32),
            pltpu.VMEM((EMB_SIZE, _GW), jnp.float32),
            pltpu.VMEM((EMB_SIZE, _GW), jnp.float32),
            pltpu.VMEM((_G, 128), jnp.float32),
            pltpu.VMEM((_G, 128), jnp.float32),
            pltpu.VMEM((_G, 128), jnp.float32),
            pltpu.VMEM((_G, 128), jnp.float32),
            pltpu.VMEM((16,), jnp.float32),
            pltpu.SemaphoreType.DMA,
            pltpu.SemaphoreType.DMA,
        ],
        compiler_params=pltpu.CompilerParams(
            needs_layout_passes=False,
            use_tc_tiling_on_sc=True,
        ),
    )(_glove_body)
    return k(center, outside, coocs, weighting, ceT, oeT, cb, ob)


def kernel(center, outside, coocs, weighting, center_embedding,
           outside_embedding, center_bias, outside_bias):
    parts = _glove_partials(
        center.reshape(-1), outside.reshape(-1),
        coocs.reshape(-1), weighting.reshape(-1),
        center_embedding.T, outside_embedding.T,
        center_bias.T, outside_bias.T,
    )
    return jnp.sum(parts)


# speedup vs baseline: 1.0352x; 1.0008x over previous
import functools

import jax
import jax.numpy as jnp
from jax import lax
from jax.experimental import pallas as pl
from jax.experimental.pallas import tpu as pltpu
from jax.experimental.pallas import tpu_sc as plsc

VOC_SIZE = 1000000
EMB_SIZE = 16
BATCH = 16384

_NC = 2
_NS = 16
_NW = _NC * _NS
_BPW = BATCH // _NW
_G = 8
_NG = _BPW // _G
_GW = _G * 128
_WMAX = VOC_SIZE - 128


def _win_base_vec(v):
    c = lax.shift_left(lax.shift_right_logical(v, 7), 7)
    return jnp.minimum(c, _WMAX)


def _glove_body(center_hbm, outside_hbm, coocs_hbm, w_hbm,
                ceT_hbm, oeT_hbm, cb_hbm, ob_hbm, out_hbm,
                cidx_v, oidx_v, cooc_v, wv_v,
                cewA, cewB, oewA, oewB, cbwA, cbwB, obwA, obwB,
                out_v, semA, semB, semP):
    wid = lax.axis_index("s") * _NC + lax.axis_index("c")
    base = wid * _BPW

    c1 = pltpu.make_async_copy(center_hbm.at[pl.ds(base, _BPW)], cidx_v, semP)
    c2 = pltpu.make_async_copy(outside_hbm.at[pl.ds(base, _BPW)], oidx_v, semP)
    c3 = pltpu.make_async_copy(coocs_hbm.at[pl.ds(base, _BPW)], cooc_v, semP)
    c4 = pltpu.make_async_copy(w_hbm.at[pl.ds(base, _BPW)], wv_v, semP)
    c1.start(); c2.start(); c3.start(); c4.start()
    c1.wait(); c2.wait(); c3.wait(); c4.wait()

    lane = lax.broadcasted_iota(jnp.int32, (16,), 0)
    zeros16 = jnp.zeros((16,), jnp.int32)

    def fire(cvv, cuv, l0, cew, oew, cbw, obw, sem):
        for j in range(_G):
            cv = pl.multiple_of(cvv[l0 + j], 128)
            cu = pl.multiple_of(cuv[l0 + j], 128)
            pltpu.make_async_copy(
                ceT_hbm.at[pl.ds(0, 8), pl.ds(cv, 128)],
                cew.at[pl.ds(0, 8), pl.ds(j * 128, 128)], sem).start()
            pltpu.make_async_copy(
                ceT_hbm.at[pl.ds(8, 8), pl.ds(cv, 128)],
                cew.at[pl.ds(8, 8), pl.ds(j * 128, 128)], sem).start()
            pltpu.make_async_copy(
                oeT_hbm.at[pl.ds(0, 8), pl.ds(cu, 128)],
                oew.at[pl.ds(0, 8), pl.ds(j * 128, 128)], sem).start()
            pltpu.make_async_copy(
                oeT_hbm.at[pl.ds(8, 8), pl.ds(cu, 128)],
                oew.at[pl.ds(8, 8), pl.ds(j * 128, 128)], sem).start()
            pltpu.make_async_copy(cb_hbm.at[:, pl.ds(cv, 128)],
                                  cbw.at[pl.ds(j, 1), :], sem).start()
            pltpu.make_async_copy(ob_hbm.at[:, pl.ds(cu, 128)],
                                  obw.at[pl.ds(j, 1), :], sem).start()

    def drain(cew, oew, cbw, obw, sem):
        pltpu.make_async_copy(ceT_hbm.at[:, pl.ds(0, _GW)], cew, sem).wait()
        pltpu.make_async_copy(oeT_hbm.at[:, pl.ds(0, _GW)], oew, sem).wait()
        pltpu.make_async_copy(ceT_hbm.at[pl.ds(0, _G), pl.ds(0, 128)],
                              cbw, sem).wait()
        pltpu.make_async_copy(ceT_hbm.at[pl.ds(0, _G), pl.ds(0, 128)],
                              obw, sem).wait()

    halfj = lane & 7

    def compute(i, l0, cew, oew, cbw, obw, acc):
        off16 = i * 16 + l0 + halfj
        vh = plsc.load_gather(cidx_v, [off16])
        uh = plsc.load_gather(oidx_v, [off16])
        lvh = vh - _win_base_vec(vh)
        luh = uh - _win_base_vec(uh)
        colv = halfj * 128 + lvh
        colu = halfj * 128 + luh
        ip = jnp.zeros((16,), jnp.float32)
        for e in range(EMB_SIZE):
            e16 = jnp.full((16,), e, jnp.int32)
            ce = plsc.load_gather(cew, [e16, colv])
            oe = plsc.load_gather(oew, [e16, colu])
            ip = ip + ce * oe
        cb = plsc.load_gather(cbw, [halfj, lvh])
        tb = plsc.load_gather(obw, [halfj, luh])
        cooc = plsc.load_gather(cooc_v, [off16])
        w = plsc.load_gather(wv_v, [off16])
        r = ip + cb + tb - cooc
        return acc + jnp.where(lane < 8, w * r * r, 0.0)

    vc0 = cidx_v[pl.ds(0, 16)]
    vo0 = oidx_v[pl.ds(0, 16)]
    cv0 = _win_base_vec(vc0)
    cu0 = _win_base_vec(vo0)
    fire(cv0, cu0, 0, cewA, oewA, cbwA, obwA, semA)
    fire(cv0, cu0, 8, cewB, oewB, cbwB, obwB, semB)

    def pair_body(p, carry):
        vc, vo, cv, cu, acc = carry
        pnext = jnp.minimum(p + 1, _NG // 2 - 1) * 16
        vcn = cidx_v[pl.ds(pnext, 16)]
        von = oidx_v[pl.ds(pnext, 16)]
        cvn = _win_base_vec(vcn)
        cun = _win_base_vec(von)
        drain(cewA, oewA, cbwA, obwA, semA)
        acc = compute(p, 0, cewA, oewA, cbwA, obwA, acc)

        @pl.when(p < _NG // 2 - 1)
        def _():
            fire(cvn, cun, 0, cewA, oewA, cbwA, obwA, semA)

        drain(cewB, oewB, cbwB, obwB, semB)
        acc = compute(p, 8, cewB, oewB, cbwB, obwB, acc)

        @pl.when(p < _NG // 2 - 1)
        def _():
            fire(cvn, cun, 8, cewB, oewB, cbwB, obwB, semB)

        return (vcn, von, cvn, cun, acc)

    _, _, _, _, acc = lax.fori_loop(
        0, _NG // 2, pair_body,
        (vc0, vo0, cv0, cu0, jnp.zeros((16,), jnp.float32)))

    out_v[...] = acc
    pltpu.sync_copy(out_v, out_hbm.at[wid])


def _glove_partials(center, outside, coocs, weighting, ceT, oeT, cb, ob):
    mesh = plsc.VectorSubcoreMesh(core_axis_name="c", subcore_axis_name="s")
    k = functools.partial(
        pl.kernel,
        mesh=mesh,
        out_type=jax.ShapeDtypeStruct((_NW, 16), jnp.float32),
        scratch_types=[
            pltpu.VMEM((_BPW,), jnp.int32),
            pltpu.VMEM((_BPW,), jnp.int32),
            pltpu.VMEM((_BPW,), jnp.float32),
            pltpu.VMEM((_BPW,), jnp.float32),
            pltpu.VMEM((EMB_SIZE, _GW), jnp.float32),
            pltpu.VMEM((EMB_SIZE, _GW), jnp.float32),
            pltpu.VMEM((EMB_SIZE, _GW), jnp.float32),
            pltpu.VMEM((EMB_SIZE, _GW), jnp.float32),
            pltpu.VMEM((_G, 128), jnp.float32),
            pltpu.VMEM((_G, 128), jnp.float32),
            pltpu.VMEM((_G, 128), jnp.float32),
            pltpu.VMEM((_G, 128), jnp.float32),
            pltpu.VMEM((16,), jnp.float32),
            pltpu.SemaphoreType.DMA,
            pltpu.SemaphoreType.DMA,
            pltpu.SemaphoreType.DMA,
        ],
        compiler_params=pltpu.CompilerParams(
            needs_layout_passes=False,
            use_tc_tiling_on_sc=True,
        ),
    )(_glove_body)
    return k(center, outside, coocs, weighting, ceT, oeT, cb, ob)


def kernel(center, outside, coocs, weighting, center_embedding,
           outside_embedding, center_bias, outside_bias):
    parts = _glove_partials(
        center.reshape(-1), outside.reshape(-1),
        coocs.reshape(-1), weighting.reshape(-1),
        center_embedding.T, outside_embedding.T,
        center_bias.T, outside_bias.T,
    )
    return jnp.sum(parts)
